# baseline (speedup 1.0000x reference)
import functools

import jax
import jax.numpy as jnp
from jax import lax
from jax.experimental import pallas as pl
from jax.experimental.pallas import tpu as pltpu
from jax.experimental.pallas import tpu_sc as plsc

CURVATURE = 1.0

_NC = 2
_NS = 16
_NW = _NC * _NS
_CHUNK = 128


def _sc_gather_t(parent_indices, child_indices, table_t):
    D, V = table_t.shape
    B = parent_indices.shape[0]
    b_per_w = B // _NW
    n_ch = b_per_w // _CHUNK

    mesh = plsc.VectorSubcoreMesh(core_axis_name="c", subcore_axis_name="s")

    @functools.partial(
        pl.kernel,
        out_type=[
            jax.ShapeDtypeStruct((D, B), jnp.float32),
            jax.ShapeDtypeStruct((D, B), jnp.float32),
        ],
        mesh=mesh,
        compiler_params=pltpu.CompilerParams(use_tc_tiling_on_sc=False),
        scratch_types=[
            pltpu.VMEM((b_per_w,), jnp.int32),
            pltpu.VMEM((b_per_w,), jnp.int32),
            pltpu.VMEM((D, b_per_w), jnp.float32),
            pltpu.VMEM((D, b_per_w), jnp.float32),
            pltpu.SemaphoreType.DMA,
        ],
    )
    def gather_k(pidx_hbm, cidx_hbm, table_hbm, pout_hbm, cout_hbm,
                 pidx_v, cidx_v, pbuf, cbuf, sem):
        wid = lax.axis_index("s") * _NC + lax.axis_index("c")
        base = wid * b_per_w
        pltpu.sync_copy(pidx_hbm.at[pl.ds(base, b_per_w)], pidx_v)
        pltpu.sync_copy(cidx_hbm.at[pl.ds(base, b_per_w)], cidx_v)
        copies = []
        for c in range(D):
            for j in range(n_ch):
                sl = pl.ds(j * _CHUNK, _CHUNK)
                copies.append(pltpu.async_copy(
                    table_hbm.at[c].at[pidx_v.at[sl]],
                    pbuf.at[c].at[sl], sem))
                copies.append(pltpu.async_copy(
                    table_hbm.at[c].at[cidx_v.at[sl]],
                    cbuf.at[c].at[sl], sem))
        for cp in copies:
            cp.wait()
        pltpu.sync_copy(pbuf, pout_hbm.at[:, pl.ds(base, b_per_w)])
        pltpu.sync_copy(cbuf, cout_hbm.at[:, pl.ds(base, b_per_w)])

    return gather_k(parent_indices, child_indices, table_t)


def _tc_body(p_ref, c_ref, o_ref):
    u = p_ref[...]
    v = c_ref[...]
    eps = 1e-05
    max_norm = 1 - eps

    def project(x):
        norm = jnp.sqrt(jnp.sum(x * x, axis=0, keepdims=True))
        scale = jnp.where(norm >= max_norm, max_norm / (norm + 1e-07), 1.0)
        return x * scale

    u = project(u)
    v = project(v)
    u_sq = jnp.sum(u * u, axis=0)
    v_sq = jnp.sum(v * v, axis=0)
    d_sq = jnp.sum((u - v) * (u - v), axis=0)
    q = 2.0 * d_sq / ((1.0 - u_sq) * (1.0 - v_sq) + 1e-07)
    dist = jnp.log1p(q + jnp.sqrt(q * (q + 2.0)))
    o_ref[0, 0] = jnp.sum(dist) * (CURVATURE / p_ref.shape[1])


def _tc_distance(parent_t, child_t):
    out = pl.pallas_call(
        _tc_body,
        out_shape=jax.ShapeDtypeStruct((1, 1), jnp.float32),
        out_specs=pl.BlockSpec(memory_space=pltpu.SMEM),
    )(parent_t, child_t)
    return out[0, 0]


def kernel(parent_indices, child_indices, embeddings):
    prows_t, crows_t = _sc_gather_t(parent_indices, child_indices,
                                    embeddings.T)
    return _tc_distance(prows_t, crows_t)


# speedup vs baseline: 2.6367x; 2.6367x over previous
import functools

import jax
import jax.numpy as jnp
from jax import lax
from jax.experimental import pallas as pl
from jax.experimental.pallas import tpu as pltpu
from jax.experimental.pallas import tpu_sc as plsc

CURVATURE = 1.0

_NC = 2
_NS = 16
_NW = _NC * _NS
_CHUNK = 128


def _sc_gather(parent_indices, child_indices, embeddings):
    B = parent_indices.shape[0]
    D = embeddings.shape[1]
    b_per_w = B // _NW
    n_ch = b_per_w // _CHUNK

    mesh = plsc.VectorSubcoreMesh(core_axis_name="c", subcore_axis_name="s")

    @functools.partial(
        pl.kernel,
        out_type=[
            jax.ShapeDtypeStruct((B, D), jnp.float32),
            jax.ShapeDtypeStruct((B, D), jnp.float32),
        ],
        mesh=mesh,
        compiler_params=pltpu.CompilerParams(use_tc_tiling_on_sc=False),
        scratch_types=[
            pltpu.VMEM((b_per_w,), jnp.int32),
            pltpu.VMEM((b_per_w,), jnp.int32),
            pltpu.VMEM((b_per_w, D), jnp.float32),
            pltpu.VMEM((b_per_w, D), jnp.float32),
            pltpu.SemaphoreType.DMA,
        ],
    )
    def gather_k(pidx_hbm, cidx_hbm, table_hbm, pout_hbm, cout_hbm,
                 pidx_v, cidx_v, prows_v, crows_v, sem):
        wid = lax.axis_index("s") * _NC + lax.axis_index("c")
        base = wid * b_per_w
        pltpu.sync_copy(pidx_hbm.at[pl.ds(base, b_per_w)], pidx_v)
        pltpu.sync_copy(cidx_hbm.at[pl.ds(base, b_per_w)], cidx_v)
        copies = []
        for j in range(n_ch):
            sl = pl.ds(j * _CHUNK, _CHUNK)
            copies.append(pltpu.async_copy(
                table_hbm.at[pidx_v.at[sl]], prows_v.at[sl], sem))
            copies.append(pltpu.async_copy(
                table_hbm.at[cidx_v.at[sl]], crows_v.at[sl], sem))
        for c in copies:
            c.wait()
        pltpu.sync_copy(prows_v, pout_hbm.at[pl.ds(base, b_per_w)])
        pltpu.sync_copy(crows_v, cout_hbm.at[pl.ds(base, b_per_w)])

    return gather_k(parent_indices, child_indices, embeddings)


def _tc_body(p_ref, c_ref, o_ref):
    u = p_ref[...]
    v = c_ref[...]
    eps = 1e-05
    max_norm = 1 - eps

    def project(x):
        norm = jnp.sqrt(jnp.sum(x * x, axis=-1, keepdims=True))
        scale = jnp.where(norm >= max_norm, max_norm / (norm + 1e-07), 1.0)
        return x * scale

    u = project(u)
    v = project(v)
    u_sq = jnp.sum(u * u, axis=-1)
    v_sq = jnp.sum(v * v, axis=-1)
    d_sq = jnp.sum((u - v) * (u - v), axis=-1)
    q = 2.0 * d_sq / ((1.0 - u_sq) * (1.0 - v_sq) + 1e-07)
    dist = jnp.log1p(q + jnp.sqrt(q * (q + 2.0)))
    o_ref[0, 0] = jnp.sum(dist) * (CURVATURE / p_ref.shape[0])


def _tc_distance(parent_rows, child_rows):
    out = pl.pallas_call(
        _tc_body,
        out_shape=jax.ShapeDtypeStruct((1, 1), jnp.float32),
        out_specs=pl.BlockSpec(memory_space=pltpu.SMEM),
    )(parent_rows, child_rows)
    return out[0, 0]


def kernel(parent_indices, child_indices, embeddings):
    prows, crows = _sc_gather(parent_indices, child_indices, embeddings)
    return _tc_distance(prows, crows)


# speedup vs baseline: 2.8233x; 1.0708x over previous
import functools

import jax
import jax.numpy as jnp
from jax import lax
from jax.experimental import pallas as pl
from jax.experimental.pallas import tpu as pltpu
from jax.experimental.pallas import tpu_sc as plsc

CURVATURE = 1.0

_NC = 2
_NS = 16
_NW = _NC * _NS
_CHUNK = 128


def _sc_gather(parent_indices, child_indices, embeddings):
    B = parent_indices.shape[0]
    D = embeddings.shape[1]
    b_per_w = B // _NW
    n_ch = b_per_w // _CHUNK

    mesh = plsc.VectorSubcoreMesh(core_axis_name="c", subcore_axis_name="s")

    @functools.partial(
        pl.kernel,
        out_type=[
            jax.ShapeDtypeStruct((B, D), jnp.float32),
            jax.ShapeDtypeStruct((B, D), jnp.float32),
        ],
        mesh=mesh,
        compiler_params=pltpu.CompilerParams(use_tc_tiling_on_sc=False),
        scratch_types=[
            pltpu.VMEM((b_per_w,), jnp.int32),
            pltpu.VMEM((b_per_w,), jnp.int32),
            pltpu.VMEM((b_per_w, D), jnp.float32),
            pltpu.VMEM((b_per_w, D), jnp.float32),
            pltpu.SemaphoreType.DMA,
        ],
    )
    def gather_k(pidx_hbm, cidx_hbm, table_hbm, pout_hbm, cout_hbm,
                 pidx_v, cidx_v, prows_v, crows_v, sem):
        wid = lax.axis_index("s") * _NC + lax.axis_index("c")
        base = wid * b_per_w
        pltpu.sync_copy(pidx_hbm.at[pl.ds(base, b_per_w)], pidx_v)
        pltpu.sync_copy(cidx_hbm.at[pl.ds(base, b_per_w)], cidx_v)
        copies = []
        for j in range(n_ch):
            sl = pl.ds(j * _CHUNK, _CHUNK)
            copies.append(pltpu.async_copy(
                table_hbm.at[pidx_v.at[sl]], prows_v.at[sl], sem))
            copies.append(pltpu.async_copy(
                table_hbm.at[cidx_v.at[sl]], crows_v.at[sl], sem))
        for c in copies:
            c.wait()
        pltpu.sync_copy(prows_v, pout_hbm.at[pl.ds(base, b_per_w)])
        pltpu.sync_copy(crows_v, cout_hbm.at[pl.ds(base, b_per_w)])

    return gather_k(parent_indices, child_indices, embeddings)


def _tc_body(p_ref, c_ref, o_ref):
    u = p_ref[...]
    v = c_ref[...]
    eps = 1e-05
    max_norm = 1 - eps
    n_items = p_ref.shape[0] * 8

    r = lax.broadcasted_iota(jnp.int32, (128, 8), 0)
    g = lax.broadcasted_iota(jnp.int32, (128, 8), 1)
    G = (r // 16 == g).astype(jnp.float32)

    def sums(x):
        return jnp.dot(x, G, preferred_element_type=jnp.float32)

    u_sq = sums(u * u)
    v_sq = sums(v * v)
    uv = sums(u * v)

    def scale(sq):
        norm = jnp.sqrt(sq)
        return jnp.where(norm >= max_norm, max_norm / (norm + 1e-07), 1.0)

    su = scale(u_sq)
    sv = scale(v_sq)
    u_sq = su * su * u_sq
    v_sq = sv * sv * v_sq
    uv = su * sv * uv
    d_sq = jnp.maximum(u_sq + v_sq - 2.0 * uv, 0.0)
    q = 2.0 * d_sq / ((1.0 - u_sq) * (1.0 - v_sq) + 1e-07)
    dist = jnp.log1p(q + jnp.sqrt(q * (q + 2.0)))
    o_ref[0, 0] = jnp.sum(dist) * (CURVATURE / n_items)


def _tc_distance(parent_rows, child_rows):
    B = parent_rows.shape[0]
    p2 = parent_rows.reshape(B // 8, 128)
    c2 = child_rows.reshape(B // 8, 128)
    out = pl.pallas_call(
        _tc_body,
        out_shape=jax.ShapeDtypeStruct((1, 1), jnp.float32),
        out_specs=pl.BlockSpec(memory_space=pltpu.SMEM),
    )(p2, c2)
    return out[0, 0]


def kernel(parent_indices, child_indices, embeddings):
    prows, crows = _sc_gather(parent_indices, child_indices, embeddings)
    return _tc_distance(prows, crows)


# speedup vs baseline: 2.9323x; 1.0386x over previous
import functools

import jax
import jax.numpy as jnp
from jax import lax
from jax.experimental import pallas as pl
from jax.experimental.pallas import tpu as pltpu
from jax.experimental.pallas import tpu_sc as plsc

CURVATURE = 1.0

_NC = 2
_NS = 16
_NW = _NC * _NS
_CHUNK = 128
_TBLK = 2048


def _tc_transpose_flat(table_t):
    D, V = table_t.shape
    n_blk = -(-V // _TBLK)
    r_blk = _TBLK * D // 128

    def body(t_ref, o_ref):
        x = t_ref[...]
        xT = jnp.transpose(x)
        x3 = xT.reshape(r_blk, 128 // D, D)
        for k in range(128 // D):
            o_ref[:, pl.ds(k * D, D)] = x3[:, k, :]

    out = pl.pallas_call(
        body,
        grid=(n_blk,),
        in_specs=[pl.BlockSpec((D, _TBLK), lambda i: (0, i))],
        out_specs=pl.BlockSpec((r_blk, 128), lambda i: (i, 0)),
        out_shape=jax.ShapeDtypeStruct((V * D // 128, 128), jnp.float32),
    )(table_t)
    return out.reshape(-1)


def _sc_gather_flat(parent_indices, child_indices, flat_table, D):
    B = parent_indices.shape[0]
    b_per_w = B // _NW
    e_per_w = b_per_w * D
    n_ch = e_per_w // _CHUNK

    mesh = plsc.VectorSubcoreMesh(core_axis_name="c", subcore_axis_name="s")

    @functools.partial(
        pl.kernel,
        out_type=[
            jax.ShapeDtypeStruct((B * D,), jnp.float32),
            jax.ShapeDtypeStruct((B * D,), jnp.float32),
        ],
        mesh=mesh,
        compiler_params=pltpu.CompilerParams(use_tc_tiling_on_sc=False),
        scratch_types=[
            pltpu.VMEM((b_per_w,), jnp.int32),
            pltpu.VMEM((b_per_w,), jnp.int32),
            pltpu.VMEM((e_per_w,), jnp.int32),
            pltpu.VMEM((e_per_w,), jnp.int32),
            pltpu.VMEM((e_per_w,), jnp.float32),
            pltpu.VMEM((e_per_w,), jnp.float32),
            pltpu.SemaphoreType.DMA,
        ],
    )
    def gather_k(pidx_hbm, cidx_hbm, table_hbm, pout_hbm, cout_hbm,
                 pidx_v, cidx_v, pfi_v, cfi_v, pbuf, cbuf, sem):
        wid = lax.axis_index("s") * _NC + lax.axis_index("c")
        base = wid * b_per_w
        pltpu.sync_copy(pidx_hbm.at[pl.ds(base, b_per_w)], pidx_v)
        pltpu.sync_copy(cidx_hbm.at[pl.ds(base, b_per_w)], cidx_v)
        for m in range(b_per_w // 16):
            sl_m = pl.ds(m * 16, 16)
            pv = pidx_v[sl_m] * D
            cv = cidx_v[sl_m] * D
            for c in range(D):
                pfi_v[pl.ds(c * b_per_w + m * 16, 16)] = pv + c
                cfi_v[pl.ds(c * b_per_w + m * 16, 16)] = cv + c
        copies = []
        for j in range(n_ch):
            sl = pl.ds(j * _CHUNK, _CHUNK)
            copies.append(pltpu.async_copy(
                table_hbm.at[pfi_v.at[sl]], pbuf.at[sl], sem))
            copies.append(pltpu.async_copy(
                table_hbm.at[cfi_v.at[sl]], cbuf.at[sl], sem))
        for c in copies:
            c.wait()
        pltpu.sync_copy(pbuf, pout_hbm.at[pl.ds(wid * e_per_w, e_per_w)])
        pltpu.sync_copy(cbuf, cout_hbm.at[pl.ds(wid * e_per_w, e_per_w)])

    return gather_k(parent_indices, child_indices, flat_table)


def _tc_body(p_ref, c_ref, o_ref):
    u = p_ref[...]
    v = c_ref[...]
    eps = 1e-05
    max_norm = 1 - eps
    nw_d, b_per_w = u.shape
    nw = nw_d // 16
    n_items = nw * b_per_w

    u3 = u.reshape(nw, 16, b_per_w)
    v3 = v.reshape(nw, 16, b_per_w)
    u_sq = jnp.sum(u3 * u3, axis=1)
    v_sq = jnp.sum(v3 * v3, axis=1)
    uv = jnp.sum(u3 * v3, axis=1)

    def scale(sq):
        norm = jnp.sqrt(sq)
        return jnp.where(norm >= max_norm, max_norm / (norm + 1e-07), 1.0)

    su = scale(u_sq)
    sv = scale(v_sq)
    u_sq = su * su * u_sq
    v_sq = sv * sv * v_sq
    uv = su * sv * uv
    d_sq = jnp.maximum(u_sq + v_sq - 2.0 * uv, 0.0)
    q = 2.0 * d_sq / ((1.0 - u_sq) * (1.0 - v_sq) + 1e-07)
    dist = jnp.log1p(q + jnp.sqrt(q * (q + 2.0)))
    o_ref[0, 0] = jnp.sum(dist) * (CURVATURE / n_items)


def _tc_distance(parent_flat, child_flat, D, B):
    b_per_w = B // _NW
    p2 = parent_flat.reshape(_NW * D, b_per_w)
    c2 = child_flat.reshape(_NW * D, b_per_w)
    out = pl.pallas_call(
        _tc_body,
        out_shape=jax.ShapeDtypeStruct((1, 1), jnp.float32),
        out_specs=pl.BlockSpec(memory_space=pltpu.SMEM),
    )(p2, c2)
    return out[0, 0]


def kernel(parent_indices, child_indices, embeddings):
    V, D = embeddings.shape
    B = parent_indices.shape[0]
    flat_table = _tc_transpose_flat(embeddings.T)
    pf, cf = _sc_gather_flat(parent_indices, child_indices, flat_table, D)
    return _tc_distance(pf, cf, D, B)


# speedup vs baseline: 4.2380x; 1.4453x over previous
import functools

import jax
import jax.numpy as jnp
from jax import lax
from jax.experimental import pallas as pl
from jax.experimental.pallas import tpu as pltpu
from jax.experimental.pallas import tpu_sc as plsc

CURVATURE = 1.0

_NC = 2
_NS = 16
_NW = _NC * _NS
_CHUNK = 128
_TBLK = 16384


def _tc_transpose_flat(table_t):
    D, V = table_t.shape
    n_blk = -(-V // _TBLK)
    r_blk = _TBLK * D // 128

    def body(t_ref, o_ref):
        x = t_ref[...]
        xT = jnp.transpose(x)
        x3 = xT.reshape(r_blk, 128 // D, D)
        for k in range(128 // D):
            o_ref[:, pl.ds(k * D, D)] = x3[:, k, :]

    out = pl.pallas_call(
        body,
        grid=(n_blk,),
        in_specs=[pl.BlockSpec((D, _TBLK), lambda i: (0, i))],
        out_specs=pl.BlockSpec((r_blk, 128), lambda i: (i, 0)),
        out_shape=jax.ShapeDtypeStruct((V * D // 128, 128), jnp.float32),
    )(table_t)
    return out.reshape(-1)


def _sc_gather_flat(parent_indices, child_indices, flat_table, D):
    B = parent_indices.shape[0]
    b_per_w = B // _NW
    e_per_w = b_per_w * D
    n_ch = e_per_w // _CHUNK

    mesh = plsc.VectorSubcoreMesh(core_axis_name="c", subcore_axis_name="s")

    @functools.partial(
        pl.kernel,
        out_type=[
            jax.ShapeDtypeStruct((B * D,), jnp.float32),
            jax.ShapeDtypeStruct((B * D,), jnp.float32),
        ],
        mesh=mesh,
        compiler_params=pltpu.CompilerParams(use_tc_tiling_on_sc=False),
        scratch_types=[
            pltpu.VMEM((b_per_w,), jnp.int32),
            pltpu.VMEM((b_per_w,), jnp.int32),
            pltpu.VMEM((e_per_w,), jnp.int32),
            pltpu.VMEM((e_per_w,), jnp.int32),
            pltpu.VMEM((e_per_w,), jnp.float32),
            pltpu.VMEM((e_per_w,), jnp.float32),
            pltpu.SemaphoreType.DMA,
        ],
    )
    def gather_k(pidx_hbm, cidx_hbm, table_hbm, pout_hbm, cout_hbm,
                 pidx_v, cidx_v, pfi_v, cfi_v, pbuf, cbuf, sem):
        wid = lax.axis_index("s") * _NC + lax.axis_index("c")
        base = wid * b_per_w
        pltpu.sync_copy(pidx_hbm.at[pl.ds(base, b_per_w)], pidx_v)
        pltpu.sync_copy(cidx_hbm.at[pl.ds(base, b_per_w)], cidx_v)
        for m in range(b_per_w // 16):
            sl_m = pl.ds(m * 16, 16)
            pv = pidx_v[sl_m] * D
            cv = cidx_v[sl_m] * D
            for c in range(D):
                pfi_v[pl.ds(c * b_per_w + m * 16, 16)] = pv + c
                cfi_v[pl.ds(c * b_per_w + m * 16, 16)] = cv + c
        copies = []
        for j in range(n_ch):
            sl = pl.ds(j * _CHUNK, _CHUNK)
            copies.append(pltpu.async_copy(
                table_hbm.at[pfi_v.at[sl]], pbuf.at[sl], sem))
            copies.append(pltpu.async_copy(
                table_hbm.at[cfi_v.at[sl]], cbuf.at[sl], sem))
        for c in copies:
            c.wait()
        pltpu.sync_copy(pbuf, pout_hbm.at[pl.ds(wid * e_per_w, e_per_w)])
        pltpu.sync_copy(cbuf, cout_hbm.at[pl.ds(wid * e_per_w, e_per_w)])

    return gather_k(parent_indices, child_indices, flat_table)


def _tc_body(p_ref, c_ref, o_ref):
    u = p_ref[...]
    v = c_ref[...]
    eps = 1e-05
    max_norm = 1 - eps
    nw_d, b_per_w = u.shape
    nw = nw_d // 16
    n_items = nw * b_per_w

    u3 = u.reshape(nw, 16, b_per_w)
    v3 = v.reshape(nw, 16, b_per_w)
    u_sq = jnp.sum(u3 * u3, axis=1)
    v_sq = jnp.sum(v3 * v3, axis=1)
    uv = jnp.sum(u3 * v3, axis=1)

    def scale(sq):
        norm = jnp.sqrt(sq)
        return jnp.where(norm >= max_norm, max_norm / (norm + 1e-07), 1.0)

    su = scale(u_sq)
    sv = scale(v_sq)
    u_sq = su * su * u_sq
    v_sq = sv * sv * v_sq
    uv = su * sv * uv
    d_sq = jnp.maximum(u_sq + v_sq - 2.0 * uv, 0.0)
    q = 2.0 * d_sq / ((1.0 - u_sq) * (1.0 - v_sq) + 1e-07)
    dist = jnp.log1p(q + jnp.sqrt(q * (q + 2.0)))
    o_ref[0, 0] = jnp.sum(dist) * (CURVATURE / n_items)


def _tc_distance(parent_flat, child_flat, D, B):
    b_per_w = B // _NW
    p2 = parent_flat.reshape(_NW * D, b_per_w)
    c2 = child_flat.reshape(_NW * D, b_per_w)
    out = pl.pallas_call(
        _tc_body,
        out_shape=jax.ShapeDtypeStruct((1, 1), jnp.float32),
        out_specs=pl.BlockSpec(memory_space=pltpu.SMEM),
    )(p2, c2)
    return out[0, 0]


def kernel(parent_indices, child_indices, embeddings):
    V, D = embeddings.shape
    B = parent_indices.shape[0]
    flat_table = _tc_transpose_flat(embeddings.T)
    pf, cf = _sc_gather_flat(parent_indices, child_indices, flat_table, D)
    return _tc_distance(pf, cf, D, B)
